# baseline (speedup 1.0000x reference)
import jax
import jax.numpy as jnp
from jax.experimental import pallas as pl
from jax.experimental.pallas import tpu as pltpu

_NUM_LAYERS = 3
_T = 4096
_D = 512
_CHUNK = 256
_NSTEPS = _T // _CHUNK


def _body(p_now, a_now, b_now, s_now, p_pre, a_pre, b_pre, s_pre,
          W, wih, whh, bih, bhh, out_ref, acc_ref):
    step = pl.program_id(0)

    @pl.when(step == 0)
    def _init():
        acc_ref[...] = jnp.zeros_like(acc_ref)

    refs = (p_now, a_now, b_now, s_now, p_pre, a_pre, b_pre, s_pre)
    sums = jnp.concatenate(
        [jnp.sum(r[...], axis=0, keepdims=True) for r in refs], axis=0)
    acc_ref[...] += sums

    @pl.when(step == _NSTEPS - 1)
    def _epilogue():
        x0 = acc_ref[...] * (1.0 / _T)
        tgt = x0[4:8, :]
        row = jax.lax.broadcasted_iota(jnp.int32, (8, _D), 0)
        bih_v = bih[...]
        bhh_v = bhh[...]
        x = x0
        for i in range(_NUM_LAYERS):
            m = jnp.dot(x, W[i, :, :], preferred_element_type=jnp.float32)
            msum = jnp.sum(jnp.where(row >= 4, m, 0.0), axis=0, keepdims=True)
            agg = jnp.where(row < 4, msum, 0.0)
            gi = jax.lax.dot_general(agg, wih[...], (((1,), (1,)), ((), ())),
                                     preferred_element_type=jnp.float32) + bih_v
            gh = jax.lax.dot_general(x, whh[...], (((1,), (1,)), ((), ())),
                                     preferred_element_type=jnp.float32) + bhh_v
            r_g = jax.nn.sigmoid(gi[:, :_D] + gh[:, :_D])
            z_g = jax.nn.sigmoid(gi[:, _D:2 * _D] + gh[:, _D:2 * _D])
            n_g = jnp.tanh(gi[:, 2 * _D:] + r_g * gh[:, 2 * _D:])
            x = (1.0 - z_g) * n_g + z_g * x
        diff = x[0:4, :] - tgt
        d2 = jnp.sum(diff * diff, axis=1, keepdims=True)
        out_ref[...] = (10.0 / _D) * jnp.sum(d2, axis=0, keepdims=True)


def kernel(pair_now, person_1_now, person_2_now, scene_now,
           pair_pre, person_1_pre, person_2_pre, scene_pre,
           W, w_ih, w_hh, b_ih, b_hh):
    data = [x.reshape(_T, _D) for x in
            (pair_now, person_1_now, person_2_now, scene_now,
             pair_pre, person_1_pre, person_2_pre, scene_pre)]
    bih2 = b_ih.reshape(1, 3 * _D)
    bhh2 = b_hh.reshape(1, 3 * _D)

    data_spec = pl.BlockSpec((_CHUNK, _D), lambda i: (i, 0))
    full = lambda shape: pl.BlockSpec(shape, lambda i, _n=len(shape): (0,) * _n)

    out = pl.pallas_call(
        _body,
        grid=(_NSTEPS,),
        in_specs=[data_spec] * 8 + [
            full((_NUM_LAYERS, _D, _D)),
            full((3 * _D, _D)),
            full((3 * _D, _D)),
            full((1, 3 * _D)),
            full((1, 3 * _D)),
        ],
        out_specs=pl.BlockSpec((1, 1), lambda i: (0, 0)),
        out_shape=jax.ShapeDtypeStruct((1, 1), jnp.float32),
        scratch_shapes=[pltpu.VMEM((8, _D), jnp.float32)],
        compiler_params=pltpu.CompilerParams(
            dimension_semantics=("arbitrary",)),
    )(*data, W, w_ih, w_hh, bih2, bhh2)
    return out[0, 0]


# speedup vs baseline: 1.0278x; 1.0278x over previous
import jax
import jax.numpy as jnp
from jax.experimental import pallas as pl
from jax.experimental.pallas import tpu as pltpu

_NUM_LAYERS = 3
_T = 4096
_D = 512
_CHUNK = 512
_NSTEPS = _T // _CHUNK


def _body(p_now, a_now, b_now, s_now, p_pre, a_pre, b_pre, s_pre,
          W, wih, whh, bih, bhh, out_ref, acc_ref):
    step = pl.program_id(0)

    @pl.when(step == 0)
    def _init():
        acc_ref[...] = jnp.zeros_like(acc_ref)

    refs = (p_now, a_now, b_now, s_now, p_pre, a_pre, b_pre, s_pre)
    sums = jnp.concatenate(
        [jnp.sum(r[...], axis=0, keepdims=True) for r in refs], axis=0)
    acc_ref[...] += sums

    @pl.when(step == _NSTEPS - 1)
    def _epilogue():
        x0 = acc_ref[...] * (1.0 / _T)
        tgt = x0[4:8, :]
        row = jax.lax.broadcasted_iota(jnp.int32, (8, _D), 0)
        bih_v = bih[...]
        bhh_v = bhh[...]
        x = x0
        for i in range(_NUM_LAYERS):
            m = jnp.dot(x, W[i, :, :], preferred_element_type=jnp.float32)
            msum = jnp.sum(jnp.where(row >= 4, m, 0.0), axis=0, keepdims=True)
            agg = jnp.where(row < 4, msum, 0.0)
            gi = jax.lax.dot_general(agg, wih[...], (((1,), (1,)), ((), ())),
                                     preferred_element_type=jnp.float32) + bih_v
            gh = jax.lax.dot_general(x, whh[...], (((1,), (1,)), ((), ())),
                                     preferred_element_type=jnp.float32) + bhh_v
            r_g = jax.nn.sigmoid(gi[:, :_D] + gh[:, :_D])
            z_g = jax.nn.sigmoid(gi[:, _D:2 * _D] + gh[:, _D:2 * _D])
            n_g = jnp.tanh(gi[:, 2 * _D:] + r_g * gh[:, 2 * _D:])
            x = (1.0 - z_g) * n_g + z_g * x
        diff = x[0:4, :] - tgt
        d2 = jnp.sum(diff * diff, axis=1, keepdims=True)
        out_ref[...] = (10.0 / _D) * jnp.sum(d2, axis=0, keepdims=True)


def kernel(pair_now, person_1_now, person_2_now, scene_now,
           pair_pre, person_1_pre, person_2_pre, scene_pre,
           W, w_ih, w_hh, b_ih, b_hh):
    data = [x.reshape(_T, _D) for x in
            (pair_now, person_1_now, person_2_now, scene_now,
             pair_pre, person_1_pre, person_2_pre, scene_pre)]
    bih2 = b_ih.reshape(1, 3 * _D)
    bhh2 = b_hh.reshape(1, 3 * _D)

    data_spec = pl.BlockSpec((_CHUNK, _D), lambda i: (i, 0))
    full = lambda shape: pl.BlockSpec(shape, lambda i, _n=len(shape): (0,) * _n)

    out = pl.pallas_call(
        _body,
        grid=(_NSTEPS,),
        in_specs=[data_spec] * 8 + [
            full((_NUM_LAYERS, _D, _D)),
            full((3 * _D, _D)),
            full((3 * _D, _D)),
            full((1, 3 * _D)),
            full((1, 3 * _D)),
        ],
        out_specs=pl.BlockSpec((1, 1), lambda i: (0, 0)),
        out_shape=jax.ShapeDtypeStruct((1, 1), jnp.float32),
        scratch_shapes=[pltpu.VMEM((8, _D), jnp.float32)],
        compiler_params=pltpu.CompilerParams(
            dimension_semantics=("arbitrary",)),
    )(*data, W, w_ih, w_hh, bih2, bhh2)
    return out[0, 0]


# speedup vs baseline: 1.0367x; 1.0087x over previous
import jax
import jax.numpy as jnp
from jax.experimental import pallas as pl
from jax.experimental.pallas import tpu as pltpu

_NUM_LAYERS = 3
_T = 4096
_D = 512
_CHUNK = 512
_PAR = 2
_STEPS = _T // _CHUNK // _PAR


def _sum_body(p_now, a_now, b_now, s_now, p_pre, a_pre, b_pre, s_pre,
              out_ref, acc_ref):
    j = pl.program_id(1)

    @pl.when(j == 0)
    def _init():
        acc_ref[...] = jnp.zeros_like(acc_ref)

    refs = (p_now, a_now, b_now, s_now, p_pre, a_pre, b_pre, s_pre)
    sums = jnp.concatenate(
        [jnp.sum(r[...], axis=0, keepdims=True) for r in refs], axis=0)
    acc_ref[...] += sums

    @pl.when(j == _STEPS - 1)
    def _emit():
        out_ref[0, :, :] = acc_ref[...]


def _epi_body(part, W, wih, whh, bih, bhh, out_ref):
    x0 = jnp.sum(part[...], axis=0) * (1.0 / _T)
    tgt = x0[4:8, :]
    row = jax.lax.broadcasted_iota(jnp.int32, (8, _D), 0)
    bih_v = bih[...]
    bhh_v = bhh[...]
    x = x0
    for i in range(_NUM_LAYERS):
        m = jnp.dot(x, W[i, :, :], preferred_element_type=jnp.float32)
        msum = jnp.sum(jnp.where(row >= 4, m, 0.0), axis=0, keepdims=True)
        agg = jnp.where(row < 4, msum, 0.0)
        gi = jax.lax.dot_general(agg, wih[...], (((1,), (1,)), ((), ())),
                                 preferred_element_type=jnp.float32) + bih_v
        gh = jax.lax.dot_general(x, whh[...], (((1,), (1,)), ((), ())),
                                 preferred_element_type=jnp.float32) + bhh_v
        r_g = jax.nn.sigmoid(gi[:, :_D] + gh[:, :_D])
        z_g = jax.nn.sigmoid(gi[:, _D:2 * _D] + gh[:, _D:2 * _D])
        n_g = jnp.tanh(gi[:, 2 * _D:] + r_g * gh[:, 2 * _D:])
        x = (1.0 - z_g) * n_g + z_g * x
    diff = x[0:4, :] - tgt
    d2 = jnp.sum(diff * diff, axis=1, keepdims=True)
    out_ref[...] = (10.0 / _D) * jnp.sum(d2, axis=0, keepdims=True)


def kernel(pair_now, person_1_now, person_2_now, scene_now,
           pair_pre, person_1_pre, person_2_pre, scene_pre,
           W, w_ih, w_hh, b_ih, b_hh):
    data = [x.reshape(_T, _D) for x in
            (pair_now, person_1_now, person_2_now, scene_now,
             pair_pre, person_1_pre, person_2_pre, scene_pre)]
    bih2 = b_ih.reshape(1, 3 * _D)
    bhh2 = b_hh.reshape(1, 3 * _D)

    data_spec = pl.BlockSpec((_CHUNK, _D), lambda i, j: (i * _STEPS + j, 0))

    partials = pl.pallas_call(
        _sum_body,
        grid=(_PAR, _STEPS),
        in_specs=[data_spec] * 8,
        out_specs=pl.BlockSpec((1, 8, _D), lambda i, j: (i, 0, 0)),
        out_shape=jax.ShapeDtypeStruct((_PAR, 8, _D), jnp.float32),
        scratch_shapes=[pltpu.VMEM((8, _D), jnp.float32)],
        compiler_params=pltpu.CompilerParams(
            dimension_semantics=("parallel", "arbitrary")),
    )(*data)

    full = lambda shape: pl.BlockSpec(shape, lambda _n=len(shape): (0,) * _n)
    out = pl.pallas_call(
        _epi_body,
        grid=(),
        in_specs=[
            full((_PAR, 8, _D)),
            full((_NUM_LAYERS, _D, _D)),
            full((3 * _D, _D)),
            full((3 * _D, _D)),
            full((1, 3 * _D)),
            full((1, 3 * _D)),
        ],
        out_specs=full((1, 1)),
        out_shape=jax.ShapeDtypeStruct((1, 1), jnp.float32),
    )(partials, W, w_ih, w_hh, bih2, bhh2)
    return out[0, 0]
